# baseline (speedup 1.0000x reference)
import functools

import jax
import jax.numpy as jnp
from jax import lax
from jax.experimental import pallas as pl
from jax.experimental.pallas import tpu as pltpu
from jax.experimental.pallas import tpu_sc as plsc

N_NODES = 10000
N_EDGES = 320000
D = 128
L = 16

NC = 2
NS = 16
NW = NC * NS
CH = 128
K = 80
TS = K // 8
E_PAD = NW * K * CH
ROWS = 10240
RPW = ROWS // NS
CROWS = ROWS // CH
ND = 10
DRPW = ROWS // ND


def _sc_aggregate(x, src_p, dst_p, zrows):
  mesh = plsc.VectorSubcoreMesh(core_axis_name="c", subcore_axis_name="s")

  @functools.partial(
      pl.kernel,
      mesh=mesh,
      out_type=[
          jax.ShapeDtypeStruct((NC, ROWS, D), jnp.float32),
          jax.ShapeDtypeStruct((NC, NS, CROWS, CH), jnp.float32),
      ],
      scratch_types=[
          pltpu.VMEM((8, CH), jnp.int32),
          pltpu.VMEM((8, CH), jnp.int32),
          pltpu.VMEM((8, CH), jnp.int32),
          pltpu.VMEM((8, CH), jnp.int32),
          pltpu.VMEM((CH, D), jnp.float32),
          pltpu.VMEM((CH, D), jnp.float32),
          pltpu.VMEM((CROWS, CH), jnp.float32),
          pltpu.VMEM((DRPW + L,), jnp.float32),
          pltpu.VMEM_SHARED((ROWS, D), jnp.float32),
          pltpu.SemaphoreType.DMA,
          pltpu.SemaphoreType.DMA,
          pltpu.SemaphoreType.DMA,
          pltpu.SemaphoreType.DMA,
          pltpu.SemaphoreType.DMA,
          pltpu.SemaphoreType.DMA,
      ],
      compiler_params=pltpu.CompilerParams(needs_layout_passes=False),
  )
  def k(x_h, src_h, dst_h, zr_h, pout_h, cout_h,
        src_sl0, src_sl1, dst_sl0, dst_sl1, rows0, rows1, cnt_v, crec_v,
        acc, sem_is0, sem_is1, sem_id0, sem_id1, sem_g0, sem_g1):
    src_sl = (src_sl0, src_sl1)
    dst_sl = (dst_sl0, dst_sl1)
    rows = (rows0, rows1)
    sem_is = (sem_is0, sem_is1)
    sem_id = (sem_id0, sem_id1)
    sem_g = (sem_g0, sem_g1)

    cid = lax.axis_index("c")
    sid = lax.axis_index("s")
    wid = cid * NS + sid
    base = sid * RPW

    pltpu.sync_copy(zr_h, rows0)
    zh = [
        pltpu.async_copy(rows0, acc.at[pl.ds(base + r * CH, CH)], sem_g0)
        for r in range(RPW // CH)
    ]
    zero16 = jnp.zeros((L,), jnp.float32)

    def zstep(v, carry):
      row = lax.shift_right_logical(v, 3)
      col = lax.mul(lax.bitwise_and(v, 7), L)
      cnt_v[row, pl.ds(col, L)] = zero16
      return carry

    lax.fori_loop(0, CROWS * CH // L, zstep, 0)
    for h in zh:
      h.wait()

    one16 = jnp.full((L,), 1.0, jnp.float32)

    def count_slab_copy(u, b):
      w2 = sid + jnp.where(u < TS, 0, NS)
      t = lax.rem(u, TS)
      return pltpu.async_copy(
          dst_h.at[w2, pl.ds(t * 8, 8)], dst_sl[b], sem_id[b])

    def count_slab_wait(b):
      pltpu.make_async_copy(
          dst_h.at[0, pl.ds(0, 8)], dst_sl[b], sem_id[b]).wait()

    def count_rows(buf):
      def cstep(i, carry):
        for c in range(CH // L):
          dvec = buf[i, pl.ds(c * L, L)]
          i0 = lax.shift_right_logical(dvec, 7)
          i1 = lax.bitwise_and(dvec, 127)
          plsc.addupdate_scatter(cnt_v, [i0, i1], one16)
        return carry
      lax.fori_loop(0, 8, cstep, 0)

    count_slab_copy(0, 0)
    count_slab_copy(1, 1)

    def cbody(u2, carry):
      for b in range(2):
        count_slab_wait(b)
        count_rows(dst_sl[b])

        @pl.when(u2 + 1 < TS)
        def _():
          count_slab_copy(2 * u2 + 2 + b, b)

      return carry

    lax.fori_loop(0, TS, cbody, 0)

    pltpu.sync_copy(cnt_v, cout_h.at[cid, sid])
    plsc.subcore_barrier()

    def slab_copy(t, b):
      pltpu.async_copy(src_h.at[wid, pl.ds(t * 8, 8)], src_sl[b], sem_is[b])
      pltpu.async_copy(dst_h.at[wid, pl.ds(t * 8, 8)], dst_sl[b], sem_id[b])

    def slab_wait(b):
      pltpu.make_async_copy(
          src_h.at[0, pl.ds(0, 8)], src_sl[b], sem_is[b]).wait()
      pltpu.make_async_copy(
          dst_h.at[0, pl.ds(0, 8)], dst_sl[b], sem_id[b]).wait()

    def gather(sb, r, rb):
      return pltpu.async_copy(x_h.at[src_sl[sb].at[r]], rows[rb], sem_g[rb])

    slab_copy(0, 0)
    slab_copy(1, 1)

    def sbody(t2, carry):
      for sb in range(2):
        slab_wait(sb)
        for r in range(8):
          gather(sb, r, 0).wait()
          pltpu.sync_copy(rows[0], acc.at[dst_sl[sb].at[r]], add=True)

        @pl.when(t2 + 1 < TS // 2)
        def _():
          slab_copy(2 * t2 + 2 + sb, sb)

      return carry

    lax.fori_loop(0, TS // 2, sbody, 0)
    plsc.subcore_barrier()

    @pl.when(sid < ND)
    def _divide():
      def cnt_slab(t):
        return pltpu.async_copy(
            cout_h.at[cid, t, pl.ds(sid * 8, 8)],
            rows[t & 1].at[pl.ds(0, 8)], sem_g[t & 1])

      def accum(t):
        buf = rows[t & 1]

        def astep(v, carry):
          row = lax.shift_right_logical(v, 3)
          col = lax.mul(lax.bitwise_and(v, 7), L)
          cur = buf[row, pl.ds(col, L)]
          if t == 0:
            crec_v[pl.ds(v * L, L)] = cur
          else:
            crec_v[pl.ds(v * L, L)] = crec_v[pl.ds(v * L, L)] + cur
          return carry

        lax.fori_loop(0, DRPW // L, astep, 0)

      dh = {0: cnt_slab(0)}
      for t in range(NS):
        dh[t].wait()
        if t + 1 < NS:
          dh[t + 1] = cnt_slab(t + 1)
        accum(t)

      def rstep(v, carry):
        s = crec_v[pl.ds(v * L, L)]
        crec_v[pl.ds(v * L, L)] = 1.0 / jnp.maximum(s, 1.0)
        return carry

      lax.fori_loop(0, DRPW // L, rstep, 0)

      dbase = sid * DRPW

      def load_chunk(r):
        return pltpu.async_copy(
            acc.at[pl.ds(dbase + r * CH, CH)], rows[r & 1], sem_g[r & 1])

      lh = {0: load_chunk(0)}
      for r in range(DRPW // CH):
        lh[r].wait()
        if r + 1 < DRPW // CH:
          lh[r + 1] = load_chunk(r + 1)
        buf = rows[r & 1]

        def dstep(row, carry):
          rvec = crec_v[pl.ds(r * CH + row, L)]
          rec = jnp.full((L,), rvec[0], jnp.float32)
          for c in range(D // L):
            buf[row, pl.ds(c * L, L)] = buf[row, pl.ds(c * L, L)] * rec
          return carry

        lax.fori_loop(0, CH, dstep, 0)
        pltpu.sync_copy(buf, pout_h.at[cid, pl.ds(dbase + r * CH, CH)])

  return k(x, src_p, dst_p, zrows)


def _tc_finish(partials, W, b2):
  rb = 1280

  def body(p_ref, w_ref, b_ref, o_ref):
    s = p_ref[0] + p_ref[1]
    o_ref[...] = lax.dot_general(
        s, w_ref[...], (((1,), (1,)), ((), ())),
        preferred_element_type=jnp.float32) + b_ref[...]

  return pl.pallas_call(
      body,
      grid=(ROWS // rb,),
      in_specs=[
          pl.BlockSpec((NC, rb, D), lambda i: (0, i, 0)),
          pl.BlockSpec((D, D), lambda i: (0, 0)),
          pl.BlockSpec((1, D), lambda i: (0, 0)),
      ],
      out_specs=pl.BlockSpec((rb, D), lambda i: (i, 0)),
      out_shape=jax.ShapeDtypeStruct((ROWS, D), jnp.float32),
  )(partials, W, b2)


def kernel(x, edge_index, W, b):
  src = edge_index[0]
  dst = edge_index[1]
  pad = E_PAD - N_EDGES
  src_p = jnp.concatenate([src, jnp.zeros((pad,), jnp.int32)]).reshape(NW, K, CH)
  dst_p = jnp.concatenate(
      [dst, jnp.full((pad,), N_NODES, jnp.int32)]).reshape(NW, K, CH)

  zrows = jnp.zeros((CH, D), jnp.float32)

  partials, _ = _sc_aggregate(x, src_p, dst_p, zrows)
  out = _tc_finish(partials, W, b.reshape(1, D))
  return out[:N_NODES]


# speedup vs baseline: 1.1393x; 1.1393x over previous
import functools

import jax
import jax.numpy as jnp
from jax import lax
from jax.experimental import pallas as pl
from jax.experimental.pallas import tpu as pltpu
from jax.experimental.pallas import tpu_sc as plsc

N_NODES = 10000
N_EDGES = 320000
D = 128
L = 16

NC = 2
NS = 16
NW = NC * NS
CH = 128
K = 80
TS = K // 8
E_PAD = NW * K * CH
ROWS = 10240
RPW = ROWS // NS
CROWS = ROWS // CH


def _sc_aggregate(x, src_p, dst_p, zrows):
  mesh = plsc.VectorSubcoreMesh(core_axis_name="c", subcore_axis_name="s")

  @functools.partial(
      pl.kernel,
      mesh=mesh,
      out_type=[
          jax.ShapeDtypeStruct((NC, ROWS, D), jnp.float32),
          jax.ShapeDtypeStruct((NC, NS, CROWS, CH), jnp.float32),
      ],
      scratch_types=[
          pltpu.VMEM((8, CH), jnp.int32),
          pltpu.VMEM((8, CH), jnp.int32),
          pltpu.VMEM((8, CH), jnp.int32),
          pltpu.VMEM((8, CH), jnp.int32),
          pltpu.VMEM((CH, D), jnp.float32),
          pltpu.VMEM((CH, D), jnp.float32),
          pltpu.VMEM((CROWS, CH), jnp.float32),
          pltpu.VMEM_SHARED((ROWS, D), jnp.float32),
          pltpu.SemaphoreType.DMA,
          pltpu.SemaphoreType.DMA,
          pltpu.SemaphoreType.DMA,
          pltpu.SemaphoreType.DMA,
          pltpu.SemaphoreType.DMA,
          pltpu.SemaphoreType.DMA,
      ],
      compiler_params=pltpu.CompilerParams(needs_layout_passes=False),
  )
  def k(x_h, src_h, dst_h, zr_h, pout_h, cout_h,
        src_sl0, src_sl1, dst_sl0, dst_sl1, rows0, rows1, cnt_v,
        acc, sem_is0, sem_is1, sem_id0, sem_id1, sem_g0, sem_g1):
    src_sl = (src_sl0, src_sl1)
    dst_sl = (dst_sl0, dst_sl1)
    rows = (rows0, rows1)
    sem_is = (sem_is0, sem_is1)
    sem_id = (sem_id0, sem_id1)
    sem_g = (sem_g0, sem_g1)

    cid = lax.axis_index("c")
    sid = lax.axis_index("s")
    wid = cid * NS + sid
    base = sid * RPW

    def slab_copy(t, b):
      pltpu.async_copy(src_h.at[wid, pl.ds(t * 8, 8)], src_sl[b], sem_is[b])
      pltpu.async_copy(dst_h.at[wid, pl.ds(t * 8, 8)], dst_sl[b], sem_id[b])

    slab_copy(0, 0)
    slab_copy(1, 1)
    pltpu.sync_copy(zr_h, rows0)
    zh = [
        pltpu.async_copy(rows0, acc.at[pl.ds(base + r * CH, CH)], sem_g0)
        for r in range(RPW // CH)
    ]
    zero16 = jnp.zeros((L,), jnp.float32)

    def zstep(v, carry):
      row = lax.shift_right_logical(v, 3)
      col = lax.mul(lax.bitwise_and(v, 7), L)
      cnt_v[row, pl.ds(col, L)] = zero16
      return carry

    lax.fori_loop(0, CROWS * CH // L, zstep, 0)
    for h in zh:
      h.wait()
    plsc.subcore_barrier()

    def slab_wait(b):
      pltpu.make_async_copy(
          src_h.at[0, pl.ds(0, 8)], src_sl[b], sem_is[b]).wait()
      pltpu.make_async_copy(
          dst_h.at[0, pl.ds(0, 8)], dst_sl[b], sem_id[b]).wait()

    def gather(sb, r, rb):
      return pltpu.async_copy(x_h.at[src_sl[sb].at[r]], rows[rb], sem_g[rb])

    one16 = jnp.full((L,), 1.0, jnp.float32)

    def count_row(buf, r):
      for c in range(CH // L):
        dvec = buf[r, pl.ds(c * L, L)]
        i0 = lax.shift_right_logical(dvec, 7)
        i1 = lax.bitwise_and(dvec, 127)
        plsc.addupdate_scatter(cnt_v, [i0, i1], one16)

    def sbody(t2, carry):
      slab_wait(0)
      g = {(0, 0): gather(0, 0, 0)}
      for sb in range(2):
        for r in range(8):
          rb = r & 1
          g[(sb, r)].wait()
          if r < 7:
            g[(sb, r + 1)] = gather(sb, r + 1, 1 - rb)
          elif sb == 0:
            slab_wait(1)
            g[(1, 0)] = gather(1, 0, 1 - rb)
          pltpu.sync_copy(rows[rb], acc.at[dst_sl[sb].at[r]], add=True)
          count_row(dst_sl[sb], r)
          if r == 7:
            @pl.when(t2 + 1 < TS // 2)
            def _():
              slab_copy(2 * t2 + 2 + sb, sb)

      return carry

    lax.fori_loop(0, TS // 2, sbody, 0)

    pltpu.sync_copy(cnt_v, cout_h.at[cid, sid])
    plsc.subcore_barrier()

    for r in range(RPW // CH):
      pltpu.sync_copy(acc.at[pl.ds(base + r * CH, CH)],
                      pout_h.at[cid, pl.ds(base + r * CH, CH)])

  return k(x, src_p, dst_p, zrows)


def _tc_finish(partials, counts, W, b2):
  def body(p_ref, c_ref, w_ref, b_ref, o_ref):
    s = p_ref[0] + p_ref[1]
    c = jnp.sum(c_ref[...], axis=(0, 1))
    c3 = jnp.broadcast_to(c[:, :, None], (CROWS, CH, D))
    c2 = jnp.reshape(c3, (ROWS, D))
    h = s / jnp.maximum(c2, 1.0)
    o_ref[...] = lax.dot_general(
        h, w_ref[...], (((1,), (1,)), ((), ())),
        preferred_element_type=jnp.float32) + b_ref[...]

  return pl.pallas_call(
      body,
      out_shape=jax.ShapeDtypeStruct((ROWS, D), jnp.float32),
  )(partials, counts, W, b2)


def kernel(x, edge_index, W, b):
  src = edge_index[0]
  dst = edge_index[1]
  pad = E_PAD - N_EDGES
  src_p = jnp.concatenate([src, jnp.zeros((pad,), jnp.int32)]).reshape(NW, K, CH)
  dst_p = jnp.concatenate(
      [dst, jnp.full((pad,), N_NODES, jnp.int32)]).reshape(NW, K, CH)

  zrows = jnp.zeros((CH, D), jnp.float32)

  partials, counts = _sc_aggregate(x, src_p, dst_p, zrows)
  out = _tc_finish(partials, counts, W, b.reshape(1, D))
  return out[:N_NODES]
